# baseline (speedup 1.0000x reference)
import functools

import jax
import jax.numpy as jnp
from jax import lax
from jax.experimental import pallas as pl
from jax.experimental.pallas import tpu as pltpu
from jax.experimental.pallas import tpu_sc as plsc

D_MODEL = 64
SCALE = 8.0
LANES = 16
LB = 128
FT = 8
FI = 8


@functools.cache
def _make_gather(seq: int, batch: int, vocab: int):
    info = plsc.get_sparse_core_info()
    NC, NS = info.num_cores, info.num_subcores
    NW = NC * NS
    n_bt = batch // LB
    n_units = seq * n_bt
    assert n_units % (2 * NW) == 0, (seq, batch)
    units_per_w = n_units // NW

    mesh = plsc.VectorSubcoreMesh(core_axis_name="c", subcore_axis_name="s")

    @functools.partial(
        pl.kernel,
        mesh=mesh,
        compiler_params=pltpu.CompilerParams(
            use_tc_tiling_on_sc=False, needs_layout_passes=False
        ),
        out_type=jax.ShapeDtypeStruct((seq, FT, n_bt, FI, LB), jnp.float32),
        scratch_types=[
            pltpu.VMEM((LB,), jnp.int32),
            pltpu.VMEM((LB,), jnp.int32),
            pltpu.VMEM((LB, D_MODEL), jnp.float32),
            pltpu.VMEM((LB, D_MODEL), jnp.float32),
            pltpu.VMEM((FT, FI, LB), jnp.float32),
            pltpu.VMEM((FT, FI, LB), jnp.float32),
            pltpu.SemaphoreType.DMA,
            pltpu.SemaphoreType.DMA,
            pltpu.SemaphoreType.DMA,
            pltpu.SemaphoreType.DMA,
        ],
    )
    def gather_kernel(xt_hbm, table_hbm, out_hbm, idx0, idx1, src0, src1,
                      dst0, dst1, gsem0, gsem1, osem0, osem1):
        wid = lax.axis_index("s") * NC + lax.axis_index("c")
        u0 = wid * units_per_w
        idxs = (idx0, idx1)
        srcs = (src0, src1)
        dsts = (dst0, dst1)
        gsems = (gsem0, gsem1)
        osems = (osem0, osem1)
        lane = jnp.arange(LANES, dtype=jnp.int32)

        def start_gather(n, b):
            u = u0 + n
            s = u // n_bt
            bt = u % n_bt
            pltpu.sync_copy(xt_hbm.at[s, pl.ds(bt * LB, LB)], idxs[b])
            pltpu.async_copy(table_hbm.at[idxs[b]], srcs[b], gsems[b])

        def wait_gather(b):
            pltpu.make_async_copy(
                table_hbm.at[pl.ds(0, LB)], srcs[b], gsems[b]
            ).wait()

        def wait_out(b):
            pltpu.make_async_copy(
                dsts[b], out_hbm.at[0, :, 0], osems[b]
            ).wait()

        def transpose_scale(b):
            src, dst = srcs[b], dsts[b]
            for f in range(D_MODEL):
                col = jnp.full((LANES,), f, jnp.int32)
                for b0 in range(0, LB, LANES):
                    vec = plsc.load_gather(src, [lane + b0, col])
                    dst[f // FI, f % FI, pl.ds(b0, LANES)] = vec * SCALE

        def start_out(n, b):
            u = u0 + n
            s = u // n_bt
            bt = u % n_bt
            pltpu.async_copy(dsts[b], out_hbm.at[s, :, bt], osems[b])

        start_gather(0, 0)

        def outer(m2, carry):
            for b in range(2):
                n = m2 * 2 + b
                if b == 0:
                    @pl.when(m2 >= 1)
                    def _():
                        wait_out(1 - b)
                    start_gather(n + 1, 1 - b)
                else:
                    @pl.when(m2 <= units_per_w // 2 - 2)
                    def _():
                        wait_out(1 - b)
                        start_gather(n + 1, 1 - b)
                wait_gather(b)
                transpose_scale(b)
                start_out(n, b)
            return carry

        lax.fori_loop(0, units_per_w // 2, outer, None)
        wait_out(0)
        wait_out(1)

    return gather_kernel


def kernel(x, table):
    s0, s1 = x.shape
    vocab = table.shape[0]
    xt = jnp.transpose(x).astype(jnp.int32)
    out5 = _make_gather(s1, s0, vocab)(xt, table)
    return out5.transpose(2, 4, 0, 1, 3).reshape(s0, s1, D_MODEL)


# speedup vs baseline: 1.9291x; 1.9291x over previous
import functools

import jax
import jax.numpy as jnp
from jax import lax
from jax.experimental import pallas as pl
from jax.experimental.pallas import tpu as pltpu
from jax.experimental.pallas import tpu_sc as plsc

D_MODEL = 64
SCALE = 8.0
LANES = 16
LB = 128
FT = 8
TILE_E = FT * 128


@functools.cache
def _make_gather(seq: int, batch: int, vocab: int):
    info = plsc.get_sparse_core_info()
    NC, NS = info.num_cores, info.num_subcores
    NW = NC * NS
    n_bt = batch // LB
    assert n_bt == NW and seq % 2 == 0, (seq, batch)

    mesh = plsc.VectorSubcoreMesh(core_axis_name="c", subcore_axis_name="s")

    @functools.partial(
        pl.kernel,
        mesh=mesh,
        compiler_params=pltpu.CompilerParams(
            use_tc_tiling_on_sc=False, needs_layout_passes=False
        ),
        out_type=jax.ShapeDtypeStruct((seq, FT, n_bt, TILE_E), jnp.float32),
        scratch_types=[
            pltpu.VMEM((seq, LB), jnp.int32),
            pltpu.VMEM((LB, D_MODEL), jnp.float32),
            pltpu.VMEM((LB, D_MODEL), jnp.float32),
            pltpu.VMEM((FT * TILE_E,), jnp.float32),
            pltpu.VMEM((FT * TILE_E,), jnp.float32),
            pltpu.SemaphoreType.DMA,
            pltpu.SemaphoreType.DMA,
            pltpu.SemaphoreType.DMA,
            pltpu.SemaphoreType.DMA,
        ],
    )
    def gather_kernel(xt_hbm, table_hbm, out_hbm, idx_all, src0, src1,
                      dst0, dst1, gsem0, gsem1, osem0, osem1):
        wid = lax.axis_index("s") * NC + lax.axis_index("c")
        srcs = (src0, src1)
        dsts = (dst0, dst1)
        gsems = (gsem0, gsem1)
        osems = (osem0, osem1)
        lane = jax.lax.iota(jnp.int32, LANES)
        perms = [(lane + k) % LANES for k in range(LANES)]
        dst_base = [perms[k] * LB + lane for k in range(LANES)]

        def start_gather(n, b):
            pltpu.async_copy(table_hbm.at[idx_all.at[n]], srcs[b], gsems[b])

        def wait_gather(b):
            pltpu.make_async_copy(
                table_hbm.at[pl.ds(0, LB)], srcs[b], gsems[b]
            ).wait()

        def wait_out(b):
            for ft in range(FT):
                pltpu.make_async_copy(
                    dsts[b].at[pl.ds(ft * TILE_E, TILE_E)],
                    out_hbm.at[0, ft, 0],
                    osems[b],
                ).wait()

        def transpose_scale(b):
            src, dst = srcs[b], dsts[b]

            def b_grp(i, carry):
                b0 = i * LANES
                row_idx = lane + b0

                def f_grp(j, carry2):
                    f0 = j * LANES
                    base = f0 * LB + b0
                    for k in range(LANES):
                        vec = plsc.load_gather(src, [row_idx, perms[k] + f0])
                        plsc.store_scatter(
                            dst, [dst_base[k] + base], vec * SCALE
                        )
                    return carry2

                lax.fori_loop(0, D_MODEL // LANES, f_grp, None)
                return carry

            lax.fori_loop(0, LB // LANES, b_grp, None)

        def start_out(n, b):
            for ft in range(FT):
                pltpu.async_copy(
                    dsts[b].at[pl.ds(ft * TILE_E, TILE_E)],
                    out_hbm.at[n, ft, wid],
                    osems[b],
                )

        pltpu.sync_copy(xt_hbm.at[:, pl.ds(wid * LB, LB)], idx_all)
        start_gather(0, 0)

        def outer(m2, carry):
            for b in range(2):
                n = m2 * 2 + b
                if b == 0:
                    @pl.when(m2 >= 1)
                    def _():
                        wait_out(1 - b)
                    start_gather(n + 1, 1 - b)
                else:
                    @pl.when(m2 <= seq // 2 - 2)
                    def _():
                        wait_out(1 - b)
                        start_gather(n + 1, 1 - b)
                wait_gather(b)
                transpose_scale(b)
                start_out(n, b)
            return carry

        lax.fori_loop(0, seq // 2, outer, None)
        wait_out(0)
        wait_out(1)

    return gather_kernel


def kernel(x, table):
    s0, s1 = x.shape
    vocab = table.shape[0]
    xt = jnp.transpose(x).astype(jnp.int32)
    out5 = _make_gather(s1, s0, vocab)(xt, table)
    return (
        out5.reshape(s1, FT, s0 // LB, FT, LB)
        .transpose(2, 4, 0, 1, 3)
        .reshape(s0, s1, D_MODEL)
    )


# speedup vs baseline: 2.7932x; 1.4479x over previous
import functools

import jax
import jax.numpy as jnp
from jax import lax
from jax.experimental import pallas as pl
from jax.experimental.pallas import tpu as pltpu
from jax.experimental.pallas import tpu_sc as plsc

D_MODEL = 64
SCALE = 8.0
LANES = 16
LB = 128
FT = 8
TILE_E = FT * 128


@functools.cache
def _make_gather(seq: int, batch: int, vocab: int):
    info = plsc.get_sparse_core_info()
    NC, NS = info.num_cores, info.num_subcores
    NW = NC * NS
    n_bt = batch // LB
    assert n_bt == NW and seq % 2 == 0, (seq, batch)

    mesh = plsc.VectorSubcoreMesh(core_axis_name="c", subcore_axis_name="s")

    @functools.partial(
        pl.kernel,
        mesh=mesh,
        compiler_params=pltpu.CompilerParams(
            use_tc_tiling_on_sc=False, needs_layout_passes=False
        ),
        out_type=jax.ShapeDtypeStruct((seq, FT, n_bt, TILE_E), jnp.float32),
        scratch_types=[
            pltpu.VMEM((seq, LB), jnp.int32),
            pltpu.VMEM((LB, D_MODEL), jnp.float32),
            pltpu.VMEM((LB, D_MODEL), jnp.float32),
            pltpu.VMEM((FT * TILE_E,), jnp.float32),
            pltpu.VMEM((FT * TILE_E,), jnp.float32),
            pltpu.SemaphoreType.DMA,
            pltpu.SemaphoreType.DMA,
            pltpu.SemaphoreType.DMA,
            pltpu.SemaphoreType.DMA,
        ],
    )
    def gather_kernel(xt_hbm, table_hbm, out_hbm, idx_all, src0, src1,
                      dst0, dst1, gsem0, gsem1, osem0, osem1):
        wid = lax.axis_index("s") * NC + lax.axis_index("c")
        srcs = (src0, src1)
        dsts = (dst0, dst1)
        gsems = (gsem0, gsem1)
        osems = (osem0, osem1)
        lane = jax.lax.iota(jnp.int32, LANES)
        perms = [(lane + k) % LANES for k in range(LANES)]
        src_base = [lane * D_MODEL + perms[k] for k in range(LANES)]
        dst_base = [perms[k] * LB + lane for k in range(LANES)]

        def start_gather(n, b):
            pltpu.async_copy(table_hbm.at[idx_all.at[n]], srcs[b], gsems[b])

        def wait_gather(b):
            pltpu.make_async_copy(
                table_hbm.at[pl.ds(0, LB)], srcs[b], gsems[b]
            ).wait()

        def wait_out(b):
            for ft in range(FT):
                pltpu.make_async_copy(
                    dsts[b].at[pl.ds(ft * TILE_E, TILE_E)],
                    out_hbm.at[0, ft, 0],
                    osems[b],
                ).wait()

        def transpose_scale(b):
            src, dst = srcs[b], dsts[b]
            n_f = D_MODEL // LANES

            @plsc.parallel_loop(0, (LB // LANES) * n_f, unroll=2)
            def _(t):
                b0 = (t // n_f) * LANES
                f0 = (t % n_f) * LANES
                row_idx = lane + b0
                dbase = f0 * LB + b0
                for k in range(LANES):
                    vec = plsc.load_gather(src, [row_idx, perms[k] + f0])
                    plsc.store_scatter(dst, [dst_base[k] + dbase], vec * SCALE)

        def start_out(n, b):
            for ft in range(FT):
                pltpu.async_copy(
                    dsts[b].at[pl.ds(ft * TILE_E, TILE_E)],
                    out_hbm.at[n, ft, wid],
                    osems[b],
                )

        pltpu.sync_copy(xt_hbm.at[:, pl.ds(wid * LB, LB)], idx_all)
        start_gather(0, 0)

        def outer(m2, carry):
            for b in range(2):
                n = m2 * 2 + b
                if b == 0:
                    @pl.when(m2 >= 1)
                    def _():
                        wait_out(1 - b)
                    start_gather(n + 1, 1 - b)
                else:
                    @pl.when(m2 <= seq // 2 - 2)
                    def _():
                        wait_out(1 - b)
                        start_gather(n + 1, 1 - b)
                wait_gather(b)
                transpose_scale(b)
                start_out(n, b)
            return carry

        lax.fori_loop(0, seq // 2, outer, None)
        wait_out(0)
        wait_out(1)

    return gather_kernel


def kernel(x, table):
    s0, s1 = x.shape
    vocab = table.shape[0]
    xt = jnp.transpose(x).astype(jnp.int32)
    out5 = _make_gather(s1, s0, vocab)(xt, table)
    return (
        out5.reshape(s1, FT, s0 // LB, FT, LB)
        .transpose(2, 4, 0, 1, 3)
        .reshape(s0, s1, D_MODEL)
    )


# speedup vs baseline: 2.7938x; 1.0002x over previous
import functools

import jax
import jax.numpy as jnp
from jax import lax
from jax.experimental import pallas as pl
from jax.experimental.pallas import tpu as pltpu
from jax.experimental.pallas import tpu_sc as plsc

D_MODEL = 64
SCALE = 8.0
LANES = 16
LB = 128
FT = 8
TILE_E = FT * 128


@functools.cache
def _make_gather(seq: int, batch: int, vocab: int):
    info = plsc.get_sparse_core_info()
    NC, NS = info.num_cores, info.num_subcores
    NW = NC * NS
    n_bt = batch // LB
    assert n_bt == NW and seq % 2 == 0, (seq, batch)

    mesh = plsc.VectorSubcoreMesh(core_axis_name="c", subcore_axis_name="s")

    @functools.partial(
        pl.kernel,
        mesh=mesh,
        compiler_params=pltpu.CompilerParams(
            use_tc_tiling_on_sc=False, needs_layout_passes=False
        ),
        out_type=jax.ShapeDtypeStruct((seq, FT, n_bt, TILE_E), jnp.float32),
        scratch_types=[
            pltpu.VMEM((seq // 8, 8, LB), jnp.int32),
            pltpu.VMEM((LB, D_MODEL), jnp.float32),
            pltpu.VMEM((LB, D_MODEL), jnp.float32),
            pltpu.VMEM((FT * TILE_E,), jnp.float32),
            pltpu.VMEM((FT * TILE_E,), jnp.float32),
            pltpu.SemaphoreType.DMA,
            pltpu.SemaphoreType.DMA,
            pltpu.SemaphoreType.DMA,
            pltpu.SemaphoreType.DMA,
        ],
    )
    def gather_kernel(xt_hbm, table_hbm, out_hbm, idx_all, src0, src1,
                      dst0, dst1, gsem0, gsem1, osem0, osem1):
        wid = lax.axis_index("s") * NC + lax.axis_index("c")
        srcs = (src0, src1)
        dsts = (dst0, dst1)
        gsems = (gsem0, gsem1)
        osems = (osem0, osem1)
        lane = jax.lax.iota(jnp.int32, LANES)
        perms = [(lane + k) % LANES for k in range(LANES)]
        src_base = [lane * D_MODEL + perms[k] for k in range(LANES)]
        dst_base = [perms[k] * LB + lane for k in range(LANES)]

        def start_gather(n, b):
            pltpu.async_copy(
                table_hbm.at[idx_all.at[n // 8, n % 8]], srcs[b], gsems[b]
            )

        def wait_gather(b):
            pltpu.make_async_copy(
                table_hbm.at[pl.ds(0, LB)], srcs[b], gsems[b]
            ).wait()

        def wait_out(b):
            for ft in range(FT):
                pltpu.make_async_copy(
                    dsts[b].at[pl.ds(ft * TILE_E, TILE_E)],
                    out_hbm.at[0, ft, 0],
                    osems[b],
                ).wait()

        def transpose_scale(b):
            src, dst = srcs[b], dsts[b]
            n_f = D_MODEL // LANES

            @plsc.parallel_loop(0, (LB // LANES) * n_f, unroll=2)
            def _(t):
                b0 = (t // n_f) * LANES
                f0 = (t % n_f) * LANES
                row_idx = lane + b0
                dbase = f0 * LB + b0
                for k in range(LANES):
                    vec = plsc.load_gather(src, [row_idx, perms[k] + f0])
                    plsc.store_scatter(dst, [dst_base[k] + dbase], vec * SCALE)

        def start_out(n, b):
            for ft in range(FT):
                pltpu.async_copy(
                    dsts[b].at[pl.ds(ft * TILE_E, TILE_E)],
                    out_hbm.at[n, ft, wid],
                    osems[b],
                )

        pltpu.sync_copy(xt_hbm.at[:, wid], idx_all)
        start_gather(0, 0)

        def outer(m2, carry):
            for b in range(2):
                n = m2 * 2 + b
                if b == 0:
                    @pl.when(m2 >= 1)
                    def _():
                        wait_out(1 - b)
                    start_gather(n + 1, 1 - b)
                else:
                    @pl.when(m2 <= seq // 2 - 2)
                    def _():
                        wait_out(1 - b)
                        start_gather(n + 1, 1 - b)
                wait_gather(b)
                transpose_scale(b)
                start_out(n, b)
            return carry

        lax.fori_loop(0, seq // 2, outer, None)
        wait_out(0)
        wait_out(1)

    return gather_kernel


def kernel(x, table):
    s0, s1 = x.shape
    vocab = table.shape[0]
    x4 = (
        x.astype(jnp.int32)
        .reshape(s0 // LB, LB, s1 // 8, 8)
        .transpose(2, 0, 3, 1)
    )
    out5 = _make_gather(s1, s0, vocab)(x4, table)
    return (
        out5.reshape(s1, FT, s0 // LB, FT, LB)
        .transpose(2, 4, 0, 1, 3)
        .reshape(s0, s1, D_MODEL)
    )
